# Initial kernel scaffold; baseline (speedup 1.0000x reference)
import jax
import jax.numpy as jnp
from jax.experimental import pallas as pl
from jax.experimental.pallas import tpu as pltpu
from jax.experimental.pallas import tpu_sc as plsc

_BETA = 0.25
_TB = 1024
_KB = 1024
_GW = 128


def _dist_argmin_kernel(x_ref, e_ref, idx_ref, dsum_ref, sx_ref, rmin_ref):
    k = pl.program_id(1)
    nk = pl.num_programs(1)

    @pl.when(k == 0)
    def _():
        x0 = x_ref[...]
        sx_ref[...] = jnp.sum(x0 * x0, axis=1, keepdims=True)

    s = jax.lax.dot_general(
        x_ref[...], e_ref[...], (((1,), (1,)), ((), ())),
        preferred_element_type=jnp.float32)
    d = sx_ref[...] - 2.0 * s
    lmin = jnp.min(d, axis=1, keepdims=True)
    ids = jax.lax.broadcasted_iota(jnp.int32, d.shape, 1) + k * _KB
    lidx = jnp.min(jnp.where(d == lmin, ids, jnp.int32(2 ** 30)),
                   axis=1, keepdims=True)

    @pl.when(k == 0)
    def _():
        rmin_ref[...] = lmin
        idx_ref[...] = lidx

    @pl.when(k > 0)
    def _():
        better = lmin < rmin_ref[...]
        rmin_ref[...] = jnp.where(better, lmin, rmin_ref[...])
        idx_ref[...] = jnp.where(better, lidx, idx_ref[...])

    @pl.when(k == nk - 1)
    def _():
        dsum_ref[...] = jnp.sum(rmin_ref[...], axis=0, keepdims=True)


def _assemble_kernel(q_ref, lat_ref, out_ref):
    q = q_ref[0]
    lat = lat_ref[0]
    qt = jnp.transpose(q, (1, 0))
    out_ref[0] = lat + (qt - lat)


def _sc_gather(emb, idx_row, n_tok, d):
    mesh = plsc.VectorSubcoreMesh(core_axis_name="c", subcore_axis_name="s")

    @pl.kernel(out_type=jax.ShapeDtypeStruct((n_tok, d), emb.dtype), mesh=mesh)
    def gather_kernel(x_hbm, i_hbm, o_hbm):
        def body(i_vmem, o_vmem):
            pltpu.sync_copy(x_hbm.at[i_vmem.at[0]], o_vmem)

        pltpu.emit_pipeline(
            body,
            grid=(n_tok // _GW,),
            in_specs=[pl.BlockSpec((1, _GW), index_map=lambda i: (0, i))],
            out_specs=[pl.BlockSpec((_GW, d), index_map=lambda i: (i, 0))],
            core_axis_name=("c", "s"),
            dimension_semantics=(pltpu.PARALLEL,),
        )(i_hbm, o_hbm)

    return gather_kernel(emb, idx_row)


def _argmin_call(flat, embedding):
    n_tok, c = flat.shape
    k_codes = embedding.shape[0]
    return pl.pallas_call(
        _dist_argmin_kernel,
        grid=(n_tok // _TB, k_codes // _KB),
        in_specs=[
            pl.BlockSpec((_TB, c), lambda t, k: (t, 0)),
            pl.BlockSpec((_KB, c), lambda t, k: (k, 0)),
        ],
        out_specs=[
            pl.BlockSpec((_TB, 1), lambda t, k: (t, 0)),
            pl.BlockSpec((1, 1), lambda t, k: (t, 0)),
        ],
        out_shape=[
            jax.ShapeDtypeStruct((n_tok, 1), jnp.int32),
            jax.ShapeDtypeStruct((n_tok // _TB, 1), jnp.float32),
        ],
        scratch_shapes=[
            pltpu.VMEM((_TB, 1), jnp.float32),
            pltpu.VMEM((_TB, 1), jnp.float32),
        ],
        compiler_params=pltpu.CompilerParams(
            dimension_semantics=("arbitrary", "arbitrary")),
    )(flat, embedding)


def _assemble_call(q, lat3):
    n, c, t = lat3.shape
    return pl.pallas_call(
        _assemble_kernel,
        grid=(n,),
        in_specs=[
            pl.BlockSpec((1, t, c), lambda i: (i, 0, 0)),
            pl.BlockSpec((1, c, t), lambda i: (i, 0, 0)),
        ],
        out_specs=pl.BlockSpec((1, c, t), lambda i: (i, 0, 0)),
        out_shape=jax.ShapeDtypeStruct((n, c, t), jnp.float32),
    )(q.reshape(n, t, c), lat3)


def kernel(latents, embedding):
    n, c, h, w = latents.shape
    t = h * w
    n_tok = n * t
    lat3 = latents.reshape(n, c, t)
    flat = jnp.transpose(latents, (0, 2, 3, 1)).reshape(n_tok, c)

    idx, dsums = _argmin_call(flat, embedding)
    q = _sc_gather(embedding, idx.reshape(1, n_tok), n_tok, c)
    out3 = _assemble_call(q, lat3)

    loss = jnp.sum(dsums) / (n_tok * c)
    vq_loss = loss + _BETA * loss
    return out3.reshape(n, c, h, w), vq_loss


# speedup vs baseline: 1.0108x; 1.0108x over previous
import jax
import jax.numpy as jnp
from jax.experimental import pallas as pl
from jax.experimental.pallas import tpu as pltpu
from jax.experimental.pallas import tpu_sc as plsc

_BETA = 0.25
_TB = 1024
_KB = 1024
_GW = 128


def _dist_argmin_kernel(x_ref, x16_ref, e_ref, idx_ref, dsum_ref, sx_ref,
                        rmin_ref):
    k = pl.program_id(1)
    nk = pl.num_programs(1)

    @pl.when(k == 0)
    def _():
        x0 = x_ref[...]
        sx_ref[...] = jnp.sum(x0 * x0, axis=1, keepdims=True)

    s = jax.lax.dot_general(
        x16_ref[...], e_ref[...], (((1,), (1,)), ((), ())),
        preferred_element_type=jnp.float32)
    d = sx_ref[...] - 2.0 * s
    lmin = jnp.min(d, axis=1, keepdims=True)
    ids = jax.lax.broadcasted_iota(jnp.int32, d.shape, 1) + k * _KB
    lidx = jnp.min(jnp.where(d == lmin, ids, jnp.int32(2 ** 30)),
                   axis=1, keepdims=True)

    @pl.when(k == 0)
    def _():
        rmin_ref[...] = lmin
        idx_ref[...] = lidx

    @pl.when(k > 0)
    def _():
        better = lmin < rmin_ref[...]
        rmin_ref[...] = jnp.where(better, lmin, rmin_ref[...])
        idx_ref[...] = jnp.where(better, lidx, idx_ref[...])

    @pl.when(k == nk - 1)
    def _():
        dsum_ref[0] = jnp.sum(rmin_ref[...], axis=0, keepdims=True)


def _assemble_kernel(q_ref, lat_ref, out_ref):
    q = q_ref[0].astype(jnp.bfloat16).astype(jnp.float32)
    lat = lat_ref[0]
    qt = jnp.transpose(q, (1, 0))
    out_ref[0] = lat + (qt - lat)


def _sc_gather(emb, idx_row, n_tok, d):
    mesh = plsc.VectorSubcoreMesh(core_axis_name="c", subcore_axis_name="s")

    @pl.kernel(out_type=jax.ShapeDtypeStruct((n_tok, d), emb.dtype), mesh=mesh)
    def gather_kernel(x_hbm, i_hbm, o_hbm):
        def body(i_vmem, o_vmem):
            pltpu.sync_copy(x_hbm.at[i_vmem.at[0]], o_vmem)

        pltpu.emit_pipeline(
            body,
            grid=(n_tok // _GW,),
            in_specs=[pl.BlockSpec((1, _GW), index_map=lambda i: (0, i))],
            out_specs=[pl.BlockSpec((_GW, d), index_map=lambda i: (i, 0))],
            core_axis_name=("c", "s"),
            dimension_semantics=(pltpu.PARALLEL,),
        )(i_hbm, o_hbm)

    return gather_kernel(emb, idx_row)


def _argmin_call(flat, flat16, embedding):
    n_tok, c = flat.shape
    k_codes = embedding.shape[0]
    return pl.pallas_call(
        _dist_argmin_kernel,
        grid=(n_tok // _TB, k_codes // _KB),
        in_specs=[
            pl.BlockSpec((_TB, c), lambda t, k: (t, 0)),
            pl.BlockSpec((_TB, c), lambda t, k: (t, 0)),
            pl.BlockSpec((_KB, c), lambda t, k: (k, 0)),
        ],
        out_specs=[
            pl.BlockSpec((_TB, 1), lambda t, k: (t, 0)),
            pl.BlockSpec((1, 1, 1), lambda t, k: (t, 0, 0)),
        ],
        out_shape=[
            jax.ShapeDtypeStruct((n_tok, 1), jnp.int32),
            jax.ShapeDtypeStruct((n_tok // _TB, 1, 1), jnp.float32),
        ],
        scratch_shapes=[
            pltpu.VMEM((_TB, 1), jnp.float32),
            pltpu.VMEM((_TB, 1), jnp.float32),
        ],
        compiler_params=pltpu.CompilerParams(
            dimension_semantics=("arbitrary", "arbitrary")),
    )(flat, flat16, embedding)


def _assemble_call(q, lat3):
    n, c, t = lat3.shape
    return pl.pallas_call(
        _assemble_kernel,
        grid=(n,),
        in_specs=[
            pl.BlockSpec((1, t, c), lambda i: (i, 0, 0)),
            pl.BlockSpec((1, c, t), lambda i: (i, 0, 0)),
        ],
        out_specs=pl.BlockSpec((1, c, t), lambda i: (i, 0, 0)),
        out_shape=jax.ShapeDtypeStruct((n, c, t), jnp.float32),
    )(q.reshape(n, t, c), lat3)


def kernel(latents, embedding):
    n, c, h, w = latents.shape
    t = h * w
    n_tok = n * t
    lat3 = latents.reshape(n, c, t)
    flat = jnp.transpose(latents, (0, 2, 3, 1)).reshape(n_tok, c)
    emb16 = embedding.astype(jnp.bfloat16)
    flat16 = flat.astype(jnp.bfloat16)

    idx, dsums = _argmin_call(flat, flat16, emb16)
    q = _sc_gather(embedding, idx.reshape(1, n_tok), n_tok, c)
    out3 = _assemble_call(q, lat3)

    loss = jnp.sum(dsums) / (n_tok * c)
    vq_loss = loss + _BETA * loss
    return out3.reshape(n, c, h, w), vq_loss


# speedup vs baseline: 1.1301x; 1.1181x over previous
import jax
import jax.numpy as jnp
from jax.experimental import pallas as pl
from jax.experimental.pallas import tpu as pltpu
from jax.experimental.pallas import tpu_sc as plsc

_BETA = 0.25
_TB = 1024
_KB = 2048
_GW = 128


def _dist_argmin_kernel(x_ref, x16_ref, e_ref, idx_ref, dsum_ref, sx_ref,
                        rmin_ref):
    k = pl.program_id(1)
    nk = pl.num_programs(1)

    @pl.when(k == 0)
    def _():
        x0 = x_ref[...]
        sx_ref[...] = jnp.sum(x0 * x0, axis=1, keepdims=True)

    s = jax.lax.dot_general(
        x16_ref[...], e_ref[...], (((1,), (1,)), ((), ())),
        preferred_element_type=jnp.float32)
    d = sx_ref[...] - 2.0 * s
    lmin = jnp.min(d, axis=1, keepdims=True)
    ids = jax.lax.broadcasted_iota(jnp.int32, d.shape, 1) + k * _KB
    lidx = jnp.min(jnp.where(d == lmin, ids, jnp.int32(2 ** 30)),
                   axis=1, keepdims=True)

    @pl.when(k == 0)
    def _():
        rmin_ref[...] = lmin
        idx_ref[...] = lidx

    @pl.when(k > 0)
    def _():
        better = lmin < rmin_ref[...]
        rmin_ref[...] = jnp.where(better, lmin, rmin_ref[...])
        idx_ref[...] = jnp.where(better, lidx, idx_ref[...])

    @pl.when(k == nk - 1)
    def _():
        dsum_ref[0] = jnp.sum(rmin_ref[...], axis=0, keepdims=True)


def _assemble_kernel(q_ref, lat_ref, out_ref):
    q = q_ref[0].astype(jnp.bfloat16).astype(jnp.float32)
    lat = lat_ref[0]
    qt = jnp.transpose(q, (1, 0))
    out_ref[0] = lat + (qt - lat)


def _sc_gather(emb, idx_row, n_tok, d):
    mesh = plsc.VectorSubcoreMesh(core_axis_name="c", subcore_axis_name="s")

    @pl.kernel(out_type=jax.ShapeDtypeStruct((n_tok, d), emb.dtype), mesh=mesh)
    def gather_kernel(x_hbm, i_hbm, o_hbm):
        def body(i_vmem, o_vmem):
            pltpu.sync_copy(x_hbm.at[i_vmem.at[0]], o_vmem)

        pltpu.emit_pipeline(
            body,
            grid=(n_tok // _GW,),
            in_specs=[pl.BlockSpec((1, _GW), index_map=lambda i: (0, i))],
            out_specs=[pl.BlockSpec((_GW, d), index_map=lambda i: (i, 0))],
            core_axis_name=("c", "s"),
            dimension_semantics=(pltpu.PARALLEL,),
        )(i_hbm, o_hbm)

    return gather_kernel(emb, idx_row)


def _argmin_call(flat, flat16, embedding):
    n_tok, c = flat.shape
    k_codes = embedding.shape[0]
    return pl.pallas_call(
        _dist_argmin_kernel,
        grid=(n_tok // _TB, k_codes // _KB),
        in_specs=[
            pl.BlockSpec((_TB, c), lambda t, k: (t, 0)),
            pl.BlockSpec((_TB, c), lambda t, k: (t, 0)),
            pl.BlockSpec((_KB, c), lambda t, k: (k, 0)),
        ],
        out_specs=[
            pl.BlockSpec((_TB, 1), lambda t, k: (t, 0)),
            pl.BlockSpec((1, 1, 1), lambda t, k: (t, 0, 0)),
        ],
        out_shape=[
            jax.ShapeDtypeStruct((n_tok, 1), jnp.int32),
            jax.ShapeDtypeStruct((n_tok // _TB, 1, 1), jnp.float32),
        ],
        scratch_shapes=[
            pltpu.VMEM((_TB, 1), jnp.float32),
            pltpu.VMEM((_TB, 1), jnp.float32),
        ],
        compiler_params=pltpu.CompilerParams(
            dimension_semantics=("arbitrary", "arbitrary")),
    )(flat, flat16, embedding)


def _assemble_call(q, lat3):
    n, c, t = lat3.shape
    return pl.pallas_call(
        _assemble_kernel,
        grid=(n,),
        in_specs=[
            pl.BlockSpec((1, t, c), lambda i: (i, 0, 0)),
            pl.BlockSpec((1, c, t), lambda i: (i, 0, 0)),
        ],
        out_specs=pl.BlockSpec((1, c, t), lambda i: (i, 0, 0)),
        out_shape=jax.ShapeDtypeStruct((n, c, t), jnp.float32),
    )(q.reshape(n, t, c), lat3)


def kernel(latents, embedding):
    n, c, h, w = latents.shape
    t = h * w
    n_tok = n * t
    lat3 = latents.reshape(n, c, t)
    flat = jnp.transpose(latents, (0, 2, 3, 1)).reshape(n_tok, c)
    emb16 = embedding.astype(jnp.bfloat16)
    flat16 = flat.astype(jnp.bfloat16)

    idx, dsums = _argmin_call(flat, flat16, emb16)
    q = _sc_gather(embedding, idx.reshape(1, n_tok), n_tok, c)
    out3 = _assemble_call(q, lat3)

    loss = jnp.sum(dsums) / (n_tok * c)
    vq_loss = loss + _BETA * loss
    return out3.reshape(n, c, h, w), vq_loss


# speedup vs baseline: 1.2029x; 1.0644x over previous
import jax
import jax.numpy as jnp
from jax.experimental import pallas as pl
from jax.experimental.pallas import tpu as pltpu
from jax.experimental.pallas import tpu_sc as plsc

_BETA = 0.25
_TB = 1024
_KB = 4096
_GW = 128


def _dist_argmin_kernel(x_ref, x16_ref, e_ref, idx_ref, dsum_ref, sx_ref,
                        rmin_ref):
    k = pl.program_id(1)
    nk = pl.num_programs(1)

    @pl.when(k == 0)
    def _():
        x0 = x_ref[...]
        sx_ref[...] = jnp.sum(x0 * x0, axis=1, keepdims=True)

    s = jax.lax.dot_general(
        x16_ref[...], e_ref[...], (((1,), (1,)), ((), ())),
        preferred_element_type=jnp.float32)
    d = sx_ref[...] - 2.0 * s
    lmin = jnp.min(d, axis=1, keepdims=True)
    ids = jax.lax.broadcasted_iota(jnp.int32, d.shape, 1) + k * _KB
    lidx = jnp.min(jnp.where(d == lmin, ids, jnp.int32(2 ** 30)),
                   axis=1, keepdims=True)

    @pl.when(k == 0)
    def _():
        rmin_ref[...] = lmin
        idx_ref[...] = lidx

    @pl.when(k > 0)
    def _():
        better = lmin < rmin_ref[...]
        rmin_ref[...] = jnp.where(better, lmin, rmin_ref[...])
        idx_ref[...] = jnp.where(better, lidx, idx_ref[...])

    @pl.when(k == nk - 1)
    def _():
        dsum_ref[0] = jnp.sum(rmin_ref[...], axis=0, keepdims=True)


def _assemble_kernel(q_ref, lat_ref, out_ref):
    q = q_ref[0].astype(jnp.bfloat16).astype(jnp.float32)
    lat = lat_ref[0]
    qt = jnp.transpose(q, (1, 0))
    out_ref[0] = lat + (qt - lat)


def _sc_gather(emb, idx_row, n_tok, d):
    mesh = plsc.VectorSubcoreMesh(core_axis_name="c", subcore_axis_name="s")

    @pl.kernel(out_type=jax.ShapeDtypeStruct((n_tok, d), emb.dtype), mesh=mesh)
    def gather_kernel(x_hbm, i_hbm, o_hbm):
        def body(i_vmem, o_vmem):
            pltpu.sync_copy(x_hbm.at[i_vmem.at[0]], o_vmem)

        pltpu.emit_pipeline(
            body,
            grid=(n_tok // _GW,),
            in_specs=[pl.BlockSpec((1, _GW), index_map=lambda i: (0, i))],
            out_specs=[pl.BlockSpec((_GW, d), index_map=lambda i: (i, 0))],
            core_axis_name=("c", "s"),
            dimension_semantics=(pltpu.PARALLEL,),
        )(i_hbm, o_hbm)

    return gather_kernel(emb, idx_row)


def _argmin_call(flat, flat16, embedding):
    n_tok, c = flat.shape
    k_codes = embedding.shape[0]
    return pl.pallas_call(
        _dist_argmin_kernel,
        grid=(n_tok // _TB, k_codes // _KB),
        in_specs=[
            pl.BlockSpec((_TB, c), lambda t, k: (t, 0)),
            pl.BlockSpec((_TB, c), lambda t, k: (t, 0)),
            pl.BlockSpec((_KB, c), lambda t, k: (k, 0)),
        ],
        out_specs=[
            pl.BlockSpec((_TB, 1), lambda t, k: (t, 0)),
            pl.BlockSpec((1, 1, 1), lambda t, k: (t, 0, 0)),
        ],
        out_shape=[
            jax.ShapeDtypeStruct((n_tok, 1), jnp.int32),
            jax.ShapeDtypeStruct((n_tok // _TB, 1, 1), jnp.float32),
        ],
        scratch_shapes=[
            pltpu.VMEM((_TB, 1), jnp.float32),
            pltpu.VMEM((_TB, 1), jnp.float32),
        ],
        compiler_params=pltpu.CompilerParams(
            dimension_semantics=("arbitrary", "arbitrary")),
    )(flat, flat16, embedding)


def _assemble_call(q, lat3):
    n, c, t = lat3.shape
    return pl.pallas_call(
        _assemble_kernel,
        grid=(n,),
        in_specs=[
            pl.BlockSpec((1, t, c), lambda i: (i, 0, 0)),
            pl.BlockSpec((1, c, t), lambda i: (i, 0, 0)),
        ],
        out_specs=pl.BlockSpec((1, c, t), lambda i: (i, 0, 0)),
        out_shape=jax.ShapeDtypeStruct((n, c, t), jnp.float32),
    )(q.reshape(n, t, c), lat3)


def kernel(latents, embedding):
    n, c, h, w = latents.shape
    t = h * w
    n_tok = n * t
    lat3 = latents.reshape(n, c, t)
    flat = jnp.transpose(latents, (0, 2, 3, 1)).reshape(n_tok, c)
    emb16 = embedding.astype(jnp.bfloat16)
    flat16 = flat.astype(jnp.bfloat16)

    idx, dsums = _argmin_call(flat, flat16, emb16)
    q = _sc_gather(embedding, idx.reshape(1, n_tok), n_tok, c)
    out3 = _assemble_call(q, lat3)

    loss = jnp.sum(dsums) / (n_tok * c)
    vq_loss = loss + _BETA * loss
    return out3.reshape(n, c, h, w), vq_loss


# speedup vs baseline: 1.2307x; 1.0231x over previous
import jax
import jax.numpy as jnp
from jax.experimental import pallas as pl
from jax.experimental.pallas import tpu as pltpu
from jax.experimental.pallas import tpu_sc as plsc

_BETA = 0.25
_TB = 512
_KB = 8192
_GW = 128


def _dist_argmin_kernel(x_ref, x16_ref, e_ref, idx_ref, dsum_ref, sx_ref,
                        rmin_ref):
    k = pl.program_id(1)
    nk = pl.num_programs(1)

    @pl.when(k == 0)
    def _():
        x0 = x_ref[...]
        sx_ref[...] = jnp.sum(x0 * x0, axis=1, keepdims=True)

    s = jax.lax.dot_general(
        x16_ref[...], e_ref[...], (((1,), (1,)), ((), ())),
        preferred_element_type=jnp.float32)
    d = sx_ref[...] - 2.0 * s
    lmin = jnp.min(d, axis=1, keepdims=True)
    ids = jax.lax.broadcasted_iota(jnp.int32, d.shape, 1) + k * _KB
    lidx = jnp.min(jnp.where(d == lmin, ids, jnp.int32(2 ** 30)),
                   axis=1, keepdims=True)

    @pl.when(k == 0)
    def _():
        rmin_ref[...] = lmin
        idx_ref[...] = lidx

    @pl.when(k > 0)
    def _():
        better = lmin < rmin_ref[...]
        rmin_ref[...] = jnp.where(better, lmin, rmin_ref[...])
        idx_ref[...] = jnp.where(better, lidx, idx_ref[...])

    @pl.when(k == nk - 1)
    def _():
        dsum_ref[0] = jnp.sum(rmin_ref[...], axis=0, keepdims=True)


def _assemble_kernel(q_ref, lat_ref, out_ref):
    q = q_ref[0].astype(jnp.bfloat16).astype(jnp.float32)
    lat = lat_ref[0]
    qt = jnp.transpose(q, (1, 0))
    out_ref[0] = lat + (qt - lat)


def _sc_gather(emb, idx_row, n_tok, d):
    mesh = plsc.VectorSubcoreMesh(core_axis_name="c", subcore_axis_name="s")

    @pl.kernel(out_type=jax.ShapeDtypeStruct((n_tok, d), emb.dtype), mesh=mesh)
    def gather_kernel(x_hbm, i_hbm, o_hbm):
        def body(i_vmem, o_vmem):
            pltpu.sync_copy(x_hbm.at[i_vmem.at[0]], o_vmem)

        pltpu.emit_pipeline(
            body,
            grid=(n_tok // _GW,),
            in_specs=[pl.BlockSpec((1, _GW), index_map=lambda i: (0, i))],
            out_specs=[pl.BlockSpec((_GW, d), index_map=lambda i: (i, 0))],
            core_axis_name=("c", "s"),
            dimension_semantics=(pltpu.PARALLEL,),
        )(i_hbm, o_hbm)

    return gather_kernel(emb, idx_row)


def _argmin_call(flat, flat16, embedding):
    n_tok, c = flat.shape
    k_codes = embedding.shape[0]
    return pl.pallas_call(
        _dist_argmin_kernel,
        grid=(n_tok // _TB, k_codes // _KB),
        in_specs=[
            pl.BlockSpec((_TB, c), lambda t, k: (t, 0)),
            pl.BlockSpec((_TB, c), lambda t, k: (t, 0)),
            pl.BlockSpec((_KB, c), lambda t, k: (k, 0)),
        ],
        out_specs=[
            pl.BlockSpec((_TB, 1), lambda t, k: (t, 0)),
            pl.BlockSpec((1, 1, 1), lambda t, k: (t, 0, 0)),
        ],
        out_shape=[
            jax.ShapeDtypeStruct((n_tok, 1), jnp.int32),
            jax.ShapeDtypeStruct((n_tok // _TB, 1, 1), jnp.float32),
        ],
        scratch_shapes=[
            pltpu.VMEM((_TB, 1), jnp.float32),
            pltpu.VMEM((_TB, 1), jnp.float32),
        ],
        compiler_params=pltpu.CompilerParams(
            dimension_semantics=("arbitrary", "arbitrary")),
    )(flat, flat16, embedding)


def _assemble_call(q, lat3):
    n, c, t = lat3.shape
    return pl.pallas_call(
        _assemble_kernel,
        grid=(n,),
        in_specs=[
            pl.BlockSpec((1, t, c), lambda i: (i, 0, 0)),
            pl.BlockSpec((1, c, t), lambda i: (i, 0, 0)),
        ],
        out_specs=pl.BlockSpec((1, c, t), lambda i: (i, 0, 0)),
        out_shape=jax.ShapeDtypeStruct((n, c, t), jnp.float32),
    )(q.reshape(n, t, c), lat3)


def kernel(latents, embedding):
    n, c, h, w = latents.shape
    t = h * w
    n_tok = n * t
    lat3 = latents.reshape(n, c, t)
    flat = jnp.transpose(latents, (0, 2, 3, 1)).reshape(n_tok, c)
    emb16 = embedding.astype(jnp.bfloat16)
    flat16 = flat.astype(jnp.bfloat16)

    idx, dsums = _argmin_call(flat, flat16, emb16)
    q = _sc_gather(embedding, idx.reshape(1, n_tok), n_tok, c)
    out3 = _assemble_call(q, lat3)

    loss = jnp.sum(dsums) / (n_tok * c)
    vq_loss = loss + _BETA * loss
    return out3.reshape(n, c, h, w), vq_loss


# speedup vs baseline: 1.2593x; 1.0232x over previous
import jax
import jax.numpy as jnp
from jax.experimental import pallas as pl
from jax.experimental.pallas import tpu as pltpu
from jax.experimental.pallas import tpu_sc as plsc

_BETA = 0.25
_TB = 1024
_KB = 8192
_GW = 128


def _dist_argmin_kernel(x_ref, x16_ref, e_ref, idx_ref, dsum_ref, sx_ref,
                        rmin_ref):
    k = pl.program_id(1)
    nk = pl.num_programs(1)

    @pl.when(k == 0)
    def _():
        x0 = x_ref[...]
        sx_ref[...] = jnp.sum(x0 * x0, axis=1, keepdims=True)

    s = jax.lax.dot_general(
        x16_ref[...], e_ref[...], (((1,), (1,)), ((), ())),
        preferred_element_type=jnp.float32)
    d = sx_ref[...] - 2.0 * s
    lmin = jnp.min(d, axis=1, keepdims=True)
    ids = jax.lax.broadcasted_iota(jnp.int32, d.shape, 1) + k * _KB
    lidx = jnp.min(jnp.where(d == lmin, ids, jnp.int32(2 ** 30)),
                   axis=1, keepdims=True)

    @pl.when(k == 0)
    def _():
        rmin_ref[...] = lmin
        idx_ref[...] = lidx

    @pl.when(k > 0)
    def _():
        better = lmin < rmin_ref[...]
        rmin_ref[...] = jnp.where(better, lmin, rmin_ref[...])
        idx_ref[...] = jnp.where(better, lidx, idx_ref[...])

    @pl.when(k == nk - 1)
    def _():
        dsum_ref[0] = jnp.sum(rmin_ref[...], axis=0, keepdims=True)


def _assemble_kernel(q_ref, lat_ref, out_ref):
    q = q_ref[0].astype(jnp.bfloat16).astype(jnp.float32)
    lat = lat_ref[0]
    qt = jnp.transpose(q, (1, 0))
    out_ref[0] = lat + (qt - lat)


def _sc_gather(emb, idx_row, n_tok, d):
    mesh = plsc.VectorSubcoreMesh(core_axis_name="c", subcore_axis_name="s")

    @pl.kernel(out_type=jax.ShapeDtypeStruct((n_tok, d), emb.dtype), mesh=mesh)
    def gather_kernel(x_hbm, i_hbm, o_hbm):
        def body(i_vmem, o_vmem):
            pltpu.sync_copy(x_hbm.at[i_vmem.at[0]], o_vmem)

        pltpu.emit_pipeline(
            body,
            grid=(n_tok // _GW,),
            in_specs=[pl.BlockSpec((1, _GW), index_map=lambda i: (0, i))],
            out_specs=[pl.BlockSpec((_GW, d), index_map=lambda i: (i, 0))],
            core_axis_name=("c", "s"),
            dimension_semantics=(pltpu.PARALLEL,),
        )(i_hbm, o_hbm)

    return gather_kernel(emb, idx_row)


def _argmin_call(flat, flat16, embedding):
    n_tok, c = flat.shape
    k_codes = embedding.shape[0]
    return pl.pallas_call(
        _dist_argmin_kernel,
        grid=(n_tok // _TB, k_codes // _KB),
        in_specs=[
            pl.BlockSpec((_TB, c), lambda t, k: (t, 0)),
            pl.BlockSpec((_TB, c), lambda t, k: (t, 0)),
            pl.BlockSpec((_KB, c), lambda t, k: (k, 0)),
        ],
        out_specs=[
            pl.BlockSpec((_TB, 1), lambda t, k: (t, 0)),
            pl.BlockSpec((1, 1, 1), lambda t, k: (t, 0, 0)),
        ],
        out_shape=[
            jax.ShapeDtypeStruct((n_tok, 1), jnp.int32),
            jax.ShapeDtypeStruct((n_tok // _TB, 1, 1), jnp.float32),
        ],
        scratch_shapes=[
            pltpu.VMEM((_TB, 1), jnp.float32),
            pltpu.VMEM((_TB, 1), jnp.float32),
        ],
        compiler_params=pltpu.CompilerParams(
            dimension_semantics=("arbitrary", "arbitrary")),
    )(flat, flat16, embedding)


def _assemble_call(q, lat3):
    n, c, t = lat3.shape
    return pl.pallas_call(
        _assemble_kernel,
        grid=(n,),
        in_specs=[
            pl.BlockSpec((1, t, c), lambda i: (i, 0, 0)),
            pl.BlockSpec((1, c, t), lambda i: (i, 0, 0)),
        ],
        out_specs=pl.BlockSpec((1, c, t), lambda i: (i, 0, 0)),
        out_shape=jax.ShapeDtypeStruct((n, c, t), jnp.float32),
    )(q.reshape(n, t, c), lat3)


def kernel(latents, embedding):
    n, c, h, w = latents.shape
    t = h * w
    n_tok = n * t
    lat3 = latents.reshape(n, c, t)
    flat = jnp.transpose(latents, (0, 2, 3, 1)).reshape(n_tok, c)
    emb16 = embedding.astype(jnp.bfloat16)
    flat16 = flat.astype(jnp.bfloat16)

    idx, dsums = _argmin_call(flat, flat16, emb16)
    q = _sc_gather(embedding, idx.reshape(1, n_tok), n_tok, c)
    out3 = _assemble_call(q, lat3)

    loss = jnp.sum(dsums) / (n_tok * c)
    vq_loss = loss + _BETA * loss
    return out3.reshape(n, c, h, w), vq_loss
